# baseline (speedup 1.0000x reference)
import functools

import jax
import jax.numpy as jnp
from jax import lax
from jax.experimental import pallas as pl

B = 4096
D = 32
K = 8192
BETA = 0.25
EPS = 1e-6
KC = 2048


def _vq_body(nt, x_ref, pq_ref, codes_ref, eye_ref, quant_ref, idx_ref, loss_ref):
    x = x_ref[...]
    pq = pq_ref[...]
    bt = x.shape[0]

    nrm = jnp.sqrt(jnp.sum(pq * pq, axis=1, keepdims=True))
    u = pq / jnp.maximum(nrm, EPS)
    inv = jnp.float32(1.0) / jnp.sqrt(jnp.float32(D))
    w = u * inv

    A = w[:, :, None] - w[:, None, :]
    A2 = lax.dot_general(A, A, (((2,), (1,)), ((0,), (0,))),
                         preferred_element_type=jnp.float32)
    dotuv = jnp.sum(w, axis=1)[:, None, None]
    eye = eye_ref[...]
    Rm = eye[None, :, :] + A + A2 / (1.0 + dotuv + EPS)

    xc = jnp.sum(Rm * x[:, :, None], axis=1)
    x_sq = jnp.sum(xc * xc, axis=1, keepdims=True)

    runmin = jnp.full((bt, 1), jnp.inf, jnp.float32)
    runidx = jnp.zeros((bt,), jnp.int32)
    for c in range(K // KC):
        codesc = codes_ref[c * KC:(c + 1) * KC, :]
        dotc = lax.dot_general(xc, codesc, (((1,), (1,)), ((), ())),
                               preferred_element_type=jnp.float32)
        c_sqc = jnp.sum(codesc * codesc, axis=1)[None, :]
        distc = x_sq - 2.0 * dotc + c_sqc
        cmin = jnp.min(distc, axis=1, keepdims=True)
        kio = lax.broadcasted_iota(jnp.int32, (bt, KC), 1) + (c * KC)
        cidx = jnp.min(jnp.where(distc == cmin, kio, K), axis=1)
        better = cmin < runmin
        runmin = jnp.where(better, cmin, runmin)
        runidx = jnp.where(better[:, 0], cidx, runidx)
    idx_ref[...] = runidx

    qc = jnp.zeros((bt, D), jnp.float32)
    for c in range(K // KC):
        codesc = codes_ref[c * KC:(c + 1) * KC, :]
        kio = lax.broadcasted_iota(jnp.int32, (bt, KC), 1) + (c * KC)
        onehotc = (kio == runidx[:, None]).astype(jnp.float32)
        qc = qc + lax.dot_general(onehotc, codesc, (((1,), (0,)), ((), ())),
                                  preferred_element_type=jnp.float32)

    quant_ref[...] = lax.dot_general(Rm, qc[:, :, None], (((2,), (1,)), ((0,), (0,))),
                                     preferred_element_type=jnp.float32)[:, :, 0]

    part = jnp.sum((x - qc) ** 2).reshape(1, 1)

    @pl.when(pl.program_id(0) == 0)
    def _init():
        loss_ref[...] = jnp.zeros((1, 1), jnp.float32)

    loss_ref[...] += part

    @pl.when(pl.program_id(0) == nt - 1)
    def _finish():
        loss_ref[...] = loss_ref[...] * ((1.0 + BETA) / B)


@functools.partial(jax.jit, static_argnames=("bt",))
def _run(x, prev_q, codes2d, bt=1024):
    nt = B // bt
    quant, idx, loss = pl.pallas_call(
        functools.partial(_vq_body, nt),
        grid=(nt,),
        in_specs=[
            pl.BlockSpec((bt, D), lambda i: (i, 0)),
            pl.BlockSpec((bt, D), lambda i: (i, 0)),
            pl.BlockSpec((K, D), lambda i: (0, 0)),
            pl.BlockSpec((D, D), lambda i: (0, 0)),
        ],
        out_specs=[
            pl.BlockSpec((bt, D), lambda i: (i, 0)),
            pl.BlockSpec((bt,), lambda i: (i,)),
            pl.BlockSpec((1, 1), lambda i: (0, 0)),
        ],
        out_shape=[
            jax.ShapeDtypeStruct((B, D), jnp.float32),
            jax.ShapeDtypeStruct((B,), jnp.int32),
            jax.ShapeDtypeStruct((1, 1), jnp.float32),
        ],
    )(x, prev_q, codes2d, jnp.eye(D, dtype=jnp.float32))
    return quant, idx, loss[0, 0]


def kernel(x, prev_q, codes):
    return _run(x, prev_q, codes[0])


# speedup vs baseline: 1.2814x; 1.2814x over previous
import functools

import jax
import jax.numpy as jnp
from jax import lax
from jax.experimental import pallas as pl

B = 4096
D = 32
K = 8192
BETA = 0.25
EPS = 1e-6
KC = 2048


def _vq_body(nt, x_ref, pq_ref, codes_ref, eye_ref, quant_ref, idx_ref, loss_ref):
    x = x_ref[...]
    pq = pq_ref[...]
    bt = x.shape[0]

    nrm = jnp.sqrt(jnp.sum(pq * pq, axis=1, keepdims=True))
    u = pq / jnp.maximum(nrm, EPS)
    inv = jnp.float32(1.0) / jnp.sqrt(jnp.float32(D))
    w = u * inv

    A = w[:, :, None] - w[:, None, :]
    A2 = lax.dot_general(A, A, (((2,), (1,)), ((0,), (0,))),
                         preferred_element_type=jnp.float32)
    dotuv = jnp.sum(w, axis=1)[:, None, None]
    eye = eye_ref[...]
    Rm = eye[None, :, :] + A + A2 / (1.0 + dotuv + EPS)

    xc = jnp.sum(Rm * x[:, :, None], axis=1)
    x_sq = jnp.sum(xc * xc, axis=1, keepdims=True)

    runmin = jnp.full((bt, 1), jnp.inf, jnp.float32)
    runidx = jnp.zeros((bt,), jnp.int32)
    for c in range(K // KC):
        codesc = codes_ref[c * KC:(c + 1) * KC, :]
        dotc = lax.dot_general(xc, codesc, (((1,), (1,)), ((), ())),
                               preferred_element_type=jnp.float32)
        c_sqc = jnp.sum(codesc * codesc, axis=1)[None, :]
        distc = x_sq - 2.0 * dotc + c_sqc
        cmin = jnp.min(distc, axis=1, keepdims=True)
        kio = lax.broadcasted_iota(jnp.int32, (bt, KC), 1) + (c * KC)
        cidx = jnp.min(jnp.where(distc == cmin, kio, K), axis=1)
        better = cmin < runmin
        runmin = jnp.where(better, cmin, runmin)
        runidx = jnp.where(better[:, 0], cidx, runidx)
    idx_ref[...] = runidx

    qc = jnp.zeros((bt, D), jnp.float32)
    for c in range(K // KC):
        codesc = codes_ref[c * KC:(c + 1) * KC, :]
        kio = lax.broadcasted_iota(jnp.int32, (bt, KC), 1) + (c * KC)
        onehotc = (kio == runidx[:, None]).astype(jnp.float32)
        qc = qc + lax.dot_general(onehotc, codesc, (((1,), (0,)), ((), ())),
                                  preferred_element_type=jnp.float32)

    quant_ref[...] = lax.dot_general(Rm, qc[:, :, None], (((2,), (1,)), ((0,), (0,))),
                                     preferred_element_type=jnp.float32)[:, :, 0]

    part = jnp.sum((x - qc) ** 2).reshape(1, 1)

    @pl.when(pl.program_id(0) == 0)
    def _init():
        loss_ref[...] = jnp.zeros((1, 1), jnp.float32)

    loss_ref[...] += part

    @pl.when(pl.program_id(0) == nt - 1)
    def _finish():
        loss_ref[...] = loss_ref[...] * ((1.0 + BETA) / B)


@functools.partial(jax.jit, static_argnames=("bt",))
def _run(x, prev_q, codes2d, bt=512):
    nt = B // bt
    quant, idx, loss = pl.pallas_call(
        functools.partial(_vq_body, nt),
        grid=(nt,),
        in_specs=[
            pl.BlockSpec((bt, D), lambda i: (i, 0)),
            pl.BlockSpec((bt, D), lambda i: (i, 0)),
            pl.BlockSpec((K, D), lambda i: (0, 0)),
            pl.BlockSpec((D, D), lambda i: (0, 0)),
        ],
        out_specs=[
            pl.BlockSpec((bt, D), lambda i: (i, 0)),
            pl.BlockSpec((bt,), lambda i: (i,)),
            pl.BlockSpec((1, 1), lambda i: (0, 0)),
        ],
        out_shape=[
            jax.ShapeDtypeStruct((B, D), jnp.float32),
            jax.ShapeDtypeStruct((B,), jnp.int32),
            jax.ShapeDtypeStruct((1, 1), jnp.float32),
        ],
    )(x, prev_q, codes2d, jnp.eye(D, dtype=jnp.float32))
    return quant, idx, loss[0, 0]


def kernel(x, prev_q, codes):
    return _run(x, prev_q, codes[0])


# speedup vs baseline: 1.6757x; 1.3077x over previous
import functools

import jax
import jax.numpy as jnp
from jax import lax
from jax.experimental import pallas as pl

B = 4096
D = 32
K = 8192
BETA = 0.25
EPS = 1e-6


def _vq_body(nt, x_ref, pq_ref, codes_ref, eye_ref, quant_ref, idx_ref, loss_ref):
    x = x_ref[...]
    pq = pq_ref[...]
    codes = codes_ref[...]

    nrm = jnp.sqrt(jnp.sum(pq * pq, axis=1, keepdims=True))
    u = pq / jnp.maximum(nrm, EPS)
    inv = jnp.float32(1.0) / jnp.sqrt(jnp.float32(D))
    w = u * inv

    A = w[:, :, None] - w[:, None, :]
    A2 = lax.dot_general(A, A, (((2,), (1,)), ((0,), (0,))),
                         preferred_element_type=jnp.float32)
    c = jnp.sum(w, axis=1, keepdims=True)
    denom = 1.0 + c + EPS
    eye = eye_ref[...]
    Rm = eye[None, :, :] + A + A2 / denom[:, :, None]

    xc = jnp.sum(Rm * x[:, :, None], axis=1)

    dot = lax.dot_general(xc, codes, (((1,), (1,)), ((), ())),
                          preferred_element_type=jnp.float32)
    x_sq = jnp.sum(xc * xc, axis=1, keepdims=True)
    c_sq = jnp.sum(codes * codes, axis=1)[None, :]
    dist = x_sq - 2.0 * dot + c_sq
    kiota = lax.broadcasted_iota(jnp.int32, dist.shape, 1)
    idx = jnp.argmin(dist, axis=1).astype(jnp.int32)
    idx_ref[...] = idx

    onehot = (kiota == idx[:, None]).astype(jnp.float32)
    qc = lax.dot_general(onehot, codes, (((1,), (0,)), ((), ())),
                         preferred_element_type=jnp.float32)

    aq = jnp.sum(u * qc, axis=1, keepdims=True)
    bq = jnp.sum(qc, axis=1, keepdims=True) * inv
    avq = u * bq - inv * aq
    a2q = u * (c * bq - aq) - inv * (bq - c * aq)
    quant_ref[...] = qc + avq + a2q / denom

    part = jnp.sum((x - qc) ** 2).reshape(1, 1)

    @pl.when(pl.program_id(0) == 0)
    def _init():
        loss_ref[...] = jnp.zeros((1, 1), jnp.float32)

    loss_ref[...] += part

    @pl.when(pl.program_id(0) == nt - 1)
    def _finish():
        loss_ref[...] = loss_ref[...] * ((1.0 + BETA) / B)


@functools.partial(jax.jit, static_argnames=("bt",))
def _run(x, prev_q, codes2d, bt=512):
    nt = B // bt
    quant, idx, loss = pl.pallas_call(
        functools.partial(_vq_body, nt),
        grid=(nt,),
        in_specs=[
            pl.BlockSpec((bt, D), lambda i: (i, 0)),
            pl.BlockSpec((bt, D), lambda i: (i, 0)),
            pl.BlockSpec((K, D), lambda i: (0, 0)),
            pl.BlockSpec((D, D), lambda i: (0, 0)),
        ],
        out_specs=[
            pl.BlockSpec((bt, D), lambda i: (i, 0)),
            pl.BlockSpec((bt,), lambda i: (i,)),
            pl.BlockSpec((1, 1), lambda i: (0, 0)),
        ],
        out_shape=[
            jax.ShapeDtypeStruct((B, D), jnp.float32),
            jax.ShapeDtypeStruct((B,), jnp.int32),
            jax.ShapeDtypeStruct((1, 1), jnp.float32),
        ],
    )(x, prev_q, codes2d, jnp.eye(D, dtype=jnp.float32))
    return quant, idx, loss[0, 0]


def kernel(x, prev_q, codes):
    return _run(x, prev_q, codes[0])


# speedup vs baseline: 1.8492x; 1.1036x over previous
import functools

import jax
import jax.numpy as jnp
from jax import lax
from jax.experimental import pallas as pl

B = 4096
D = 32
K = 8192
BETA = 0.25
EPS = 1e-6


def _vq_body(nt, x_ref, pq_ref, codes_ref, eye_ref, quant_ref, idx_ref, loss_ref):
    codes = codes_ref[...]
    c_sq = jnp.sum(codes * codes, axis=1)[None, :]
    nh = 2
    bh = x_ref.shape[0] // nh
    part = jnp.zeros((1, 1), jnp.float32)
    for h in range(nh):
        sl = slice(h * bh, (h + 1) * bh)
        part = part + _vq_half(x_ref[sl, :], pq_ref[sl, :], codes, c_sq,
                               eye_ref[...], quant_ref, idx_ref, sl)

    @pl.when(pl.program_id(0) == 0)
    def _init():
        loss_ref[...] = jnp.zeros((1, 1), jnp.float32)

    loss_ref[...] += part

    @pl.when(pl.program_id(0) == nt - 1)
    def _finish():
        loss_ref[...] = loss_ref[...] * ((1.0 + BETA) / B)


def _vq_half(x, pq, codes, c_sq, eye, quant_ref, idx_ref, sl):
    nrm = jnp.sqrt(jnp.sum(pq * pq, axis=1, keepdims=True))
    u = pq / jnp.maximum(nrm, EPS)
    inv = jnp.float32(1.0) / jnp.sqrt(jnp.float32(D))
    w = u * inv

    A = w[:, :, None] - w[:, None, :]
    A2 = lax.dot_general(A, A, (((2,), (1,)), ((0,), (0,))),
                         preferred_element_type=jnp.float32)
    c = jnp.sum(w, axis=1, keepdims=True)
    denom = 1.0 + c + EPS
    Rm = eye[None, :, :] + A + A2 / denom[:, :, None]

    xc = jnp.sum(Rm * x[:, :, None], axis=1)

    dot = lax.dot_general(xc, codes, (((1,), (1,)), ((), ())),
                          preferred_element_type=jnp.float32)
    x_sq = jnp.sum(xc * xc, axis=1, keepdims=True)
    dist = x_sq - 2.0 * dot + c_sq
    kiota = lax.broadcasted_iota(jnp.int32, dist.shape, 1)
    idx = jnp.argmin(dist, axis=1).astype(jnp.int32)
    idx_ref[sl] = idx

    onehot = (kiota == idx[:, None]).astype(jnp.float32)
    qc = lax.dot_general(onehot, codes, (((1,), (0,)), ((), ())),
                         preferred_element_type=jnp.float32)

    aq = jnp.sum(u * qc, axis=1, keepdims=True)
    bq = jnp.sum(qc, axis=1, keepdims=True) * inv
    avq = u * bq - inv * aq
    a2q = u * (c * bq - aq) - inv * (bq - c * aq)
    quant_ref[sl, :] = qc + avq + a2q / denom

    return jnp.sum((x - qc) ** 2).reshape(1, 1)


@functools.partial(jax.jit, static_argnames=("bt",))
def _run(x, prev_q, codes2d, bt=512):
    nt = B // bt
    quant, idx, loss = pl.pallas_call(
        functools.partial(_vq_body, nt),
        grid=(nt,),
        in_specs=[
            pl.BlockSpec((bt, D), lambda i: (i, 0)),
            pl.BlockSpec((bt, D), lambda i: (i, 0)),
            pl.BlockSpec((K, D), lambda i: (0, 0)),
            pl.BlockSpec((D, D), lambda i: (0, 0)),
        ],
        out_specs=[
            pl.BlockSpec((bt, D), lambda i: (i, 0)),
            pl.BlockSpec((bt,), lambda i: (i,)),
            pl.BlockSpec((1, 1), lambda i: (0, 0)),
        ],
        out_shape=[
            jax.ShapeDtypeStruct((B, D), jnp.float32),
            jax.ShapeDtypeStruct((B,), jnp.int32),
            jax.ShapeDtypeStruct((1, 1), jnp.float32),
        ],
    )(x, prev_q, codes2d, jnp.eye(D, dtype=jnp.float32))
    return quant, idx, loss[0, 0]


def kernel(x, prev_q, codes):
    return _run(x, prev_q, codes[0])


# speedup vs baseline: 2.0061x; 1.0848x over previous
import functools

import jax
import jax.numpy as jnp
from jax import lax
from jax.experimental import pallas as pl

B = 4096
D = 32
K = 8192
BETA = 0.25
EPS = 1e-6


def _vq_body(nt, x_ref, pq_ref, codes_ref, eye_ref, quant_ref, idx_ref, loss_ref):
    codes = codes_ref[...]
    c_sq = jnp.sum(codes * codes, axis=1)[None, :]
    nh = 4
    bh = x_ref.shape[0] // nh
    part = jnp.zeros((1, 1), jnp.float32)
    for h in range(nh):
        sl = slice(h * bh, (h + 1) * bh)
        part = part + _vq_half(x_ref[sl, :], pq_ref[sl, :], codes, c_sq,
                               eye_ref[...], quant_ref, idx_ref, sl)

    @pl.when(pl.program_id(0) == 0)
    def _init():
        loss_ref[...] = jnp.zeros((1, 1), jnp.float32)

    loss_ref[...] += part

    @pl.when(pl.program_id(0) == nt - 1)
    def _finish():
        loss_ref[...] = loss_ref[...] * ((1.0 + BETA) / B)


def _vq_half(x, pq, codes, c_sq, eye, quant_ref, idx_ref, sl):
    nrm = jnp.sqrt(jnp.sum(pq * pq, axis=1, keepdims=True))
    u = pq / jnp.maximum(nrm, EPS)
    inv = jnp.float32(1.0) / jnp.sqrt(jnp.float32(D))
    w = u * inv

    A = w[:, :, None] - w[:, None, :]
    A2 = lax.dot_general(A, A, (((2,), (1,)), ((0,), (0,))),
                         preferred_element_type=jnp.float32)
    c = jnp.sum(w, axis=1, keepdims=True)
    denom = 1.0 + c + EPS
    Rm = eye[None, :, :] + A + A2 / denom[:, :, None]

    xc = jnp.sum(Rm * x[:, :, None], axis=1)

    dot = lax.dot_general(xc, codes, (((1,), (1,)), ((), ())),
                          preferred_element_type=jnp.float32)
    x_sq = jnp.sum(xc * xc, axis=1, keepdims=True)
    dist = x_sq - 2.0 * dot + c_sq
    kiota = lax.broadcasted_iota(jnp.int32, dist.shape, 1)
    idx = jnp.argmin(dist, axis=1).astype(jnp.int32)
    idx_ref[sl] = idx

    onehot = (kiota == idx[:, None]).astype(jnp.float32)
    qc = lax.dot_general(onehot, codes, (((1,), (0,)), ((), ())),
                         preferred_element_type=jnp.float32)

    aq = jnp.sum(u * qc, axis=1, keepdims=True)
    bq = jnp.sum(qc, axis=1, keepdims=True) * inv
    avq = u * bq - inv * aq
    a2q = u * (c * bq - aq) - inv * (bq - c * aq)
    quant_ref[sl, :] = qc + avq + a2q / denom

    return jnp.sum((x - qc) ** 2).reshape(1, 1)


@functools.partial(jax.jit, static_argnames=("bt",))
def _run(x, prev_q, codes2d, bt=512):
    nt = B // bt
    quant, idx, loss = pl.pallas_call(
        functools.partial(_vq_body, nt),
        grid=(nt,),
        in_specs=[
            pl.BlockSpec((bt, D), lambda i: (i, 0)),
            pl.BlockSpec((bt, D), lambda i: (i, 0)),
            pl.BlockSpec((K, D), lambda i: (0, 0)),
            pl.BlockSpec((D, D), lambda i: (0, 0)),
        ],
        out_specs=[
            pl.BlockSpec((bt, D), lambda i: (i, 0)),
            pl.BlockSpec((bt,), lambda i: (i,)),
            pl.BlockSpec((1, 1), lambda i: (0, 0)),
        ],
        out_shape=[
            jax.ShapeDtypeStruct((B, D), jnp.float32),
            jax.ShapeDtypeStruct((B,), jnp.int32),
            jax.ShapeDtypeStruct((1, 1), jnp.float32),
        ],
    )(x, prev_q, codes2d, jnp.eye(D, dtype=jnp.float32))
    return quant, idx, loss[0, 0]


def kernel(x, prev_q, codes):
    return _run(x, prev_q, codes[0])
